# baseline (speedup 1.0000x reference)
import functools

import jax
import jax.numpy as jnp
from jax import lax
from jax.experimental import pallas as pl
from jax.experimental.pallas import tpu as pltpu
from jax.experimental.pallas import tpu_sc as plsc

_NUM_CODES = 64
_HIDDEN = 768
_BATCH = 1024
_ROW = _NUM_CODES * _HIDDEN


@functools.cache
def _make_sc_broadcast():
    info = plsc.get_sparse_core_info()
    nw = info.num_cores * info.num_subcores
    b_per_w = _BATCH // nw
    mesh = plsc.VectorSubcoreMesh(core_axis_name="c", subcore_axis_name="s")

    @functools.partial(
        pl.kernel,
        mesh=mesh,
        out_type=jax.ShapeDtypeStruct((_BATCH, _NUM_CODES, _HIDDEN), jnp.float32),
        scratch_types=[
            pltpu.VMEM((_NUM_CODES, _HIDDEN), jnp.float32),
            pltpu.SemaphoreType.DMA,
        ],
    )
    def sc_fill(tgt_hbm, out_t, buf_t, sem):
        wid = lax.axis_index("s") * info.num_cores + lax.axis_index("c")
        base = wid * b_per_w
        pltpu.sync_copy(tgt_hbm, buf_t)
        handles = []
        for i in range(b_per_w):
            handles.append(pltpu.async_copy(buf_t, out_t.at[base + i], sem))
        for h in handles:
            h.wait()

    return sc_fill


_TC_ROWS = 8
_TC_QUEUES = 4


def _tc_body(w_hbm, o_hbm, buf, sems):
    stage = [
        pltpu.async_copy(w_hbm, buf.at[k], sems.at[_TC_QUEUES])
        for k in range(_TC_ROWS)
    ]
    for h in stage:
        h.wait()
    handles = []
    for i in range(_BATCH // _TC_ROWS):
        handles.append(
            pltpu.async_copy(
                buf,
                o_hbm.at[pl.ds(i * _TC_ROWS, _TC_ROWS)],
                sems.at[i % _TC_QUEUES],
            )
        )
    for h in handles:
        h.wait()


@functools.cache
def _make_tc_broadcast():
    return pl.pallas_call(
        _tc_body,
        in_specs=[pl.BlockSpec(memory_space=pltpu.MemorySpace.HBM)],
        out_specs=pl.BlockSpec(memory_space=pltpu.MemorySpace.HBM),
        out_shape=jax.ShapeDtypeStruct(
            (_BATCH, _NUM_CODES, _HIDDEN), jnp.float32
        ),
        scratch_shapes=[
            pltpu.VMEM((_TC_ROWS, _NUM_CODES, _HIDDEN), jnp.float32),
            pltpu.SemaphoreType.DMA((_TC_QUEUES + 1,)),
        ],
    )


def kernel(W_standard, W_target, batch_size):
    del batch_size
    out_t = _make_sc_broadcast()(W_target)
    out_s = _make_tc_broadcast()(W_standard)
    return (out_s, out_t)


# speedup vs baseline: 1.0357x; 1.0357x over previous
import functools

import jax
import jax.numpy as jnp
from jax import lax
from jax.experimental import pallas as pl
from jax.experimental.pallas import tpu as pltpu
from jax.experimental.pallas import tpu_sc as plsc

_NUM_CODES = 64
_HIDDEN = 768
_BATCH = 1024
_ROW = _NUM_CODES * _HIDDEN


@functools.cache
def _make_sc_broadcast():
    info = plsc.get_sparse_core_info()
    nw = info.num_cores * info.num_subcores
    b_per_w = _BATCH // nw
    mesh = plsc.VectorSubcoreMesh(core_axis_name="c", subcore_axis_name="s")

    @functools.partial(
        pl.kernel,
        mesh=mesh,
        out_type=jax.ShapeDtypeStruct((_BATCH, _NUM_CODES, _HIDDEN), jnp.float32),
        scratch_types=[
            pltpu.VMEM((_NUM_CODES, _HIDDEN), jnp.float32),
            pltpu.SemaphoreType.DMA,
        ],
    )
    def sc_fill(tgt_hbm, out_t, buf_t, sem):
        wid = lax.axis_index("s") * info.num_cores + lax.axis_index("c")
        base = wid * b_per_w
        pltpu.sync_copy(tgt_hbm, buf_t)
        handles = []
        for i in range(b_per_w):
            handles.append(pltpu.async_copy(buf_t, out_t.at[base + i], sem))
        for h in handles:
            h.wait()

    return sc_fill


_TC_ROWS = 8


def _tc_body(w_ref, o_ref):
    o_ref[...] = jnp.broadcast_to(w_ref[...][None], o_ref.shape)


@functools.cache
def _make_tc_broadcast():
    return pl.pallas_call(
        _tc_body,
        grid=(_BATCH // _TC_ROWS,),
        in_specs=[pl.BlockSpec((_NUM_CODES, _HIDDEN), lambda i: (0, 0))],
        out_specs=pl.BlockSpec(
            (_TC_ROWS, _NUM_CODES, _HIDDEN), lambda i: (i, 0, 0)
        ),
        out_shape=jax.ShapeDtypeStruct(
            (_BATCH, _NUM_CODES, _HIDDEN), jnp.float32
        ),
    )


def kernel(W_standard, W_target, batch_size):
    del batch_size
    out_t = _make_sc_broadcast()(W_target)
    out_s = _make_tc_broadcast()(W_standard)
    return (out_s, out_t)
